# baseline (speedup 1.0000x reference)
import functools

import jax
import jax.numpy as jnp
from jax import lax
from jax.experimental import pallas as pl
from jax.experimental.pallas import tpu as pltpu

E = 64
TOP_K = 2
G = 4


def _router(x, wr, noise, posm_s, pt_s, xbf_s, aux_ref):
    T = x.shape[0]
    C = (T * TOP_K) // E
    logits = jnp.dot(x, wr, preferred_element_type=jnp.float32)
    m = jnp.max(logits, axis=1, keepdims=True)
    p = jnp.exp(logits - m)
    p = p / jnp.sum(p, axis=1, keepdims=True)
    psum = jnp.sum(p, axis=0, keepdims=True)
    aux_ref[...] = (E * C / T) * (jnp.sum(psum, axis=1, keepdims=True) / T)

    nl = jnp.transpose(logits + noise)
    pt_s[...] = jnp.transpose(p)
    xbf_s[...] = x.astype(jnp.bfloat16)

    kbits = lax.bitcast_convert_type(nl, jnp.int32)
    skey = kbits ^ ((kbits >> 31) & jnp.int32(0x7FFFFFFF))

    def count_ge(thr):
        return jnp.sum(jnp.where(skey >= thr, 1, 0), axis=1, keepdims=True)

    def hi_body(_, carry):
        lo, hi = carry
        mid = (lo + hi) >> 1
        ge = count_ge(mid << 16) >= C
        return jnp.where(ge, mid, lo), jnp.where(ge, hi, mid)

    lo, _ = lax.fori_loop(0, 16, hi_body,
                          (jnp.full((E, 1), -32768, jnp.int32),
                           jnp.full((E, 1), 32768, jnp.int32)))
    base = lo << 16

    def lo_body(_, carry):
        lo, hi = carry
        mid = (lo + hi) >> 1
        ge = count_ge(base + mid) >= C
        return jnp.where(ge, mid, lo), jnp.where(ge, hi, mid)

    lo2, _ = lax.fori_loop(0, 16, lo_body,
                           (jnp.zeros((E, 1), jnp.int32),
                            jnp.full((E, 1), 65536, jnp.int32)))
    thr = base + lo2

    mask = skey >= thr
    acc = jnp.where(mask, 1, 0)
    for k in (1, 2, 4, 8, 16, 32, 64, 128, 256, 512, 1024):
        shifted = jnp.concatenate(
            [jnp.zeros((E, k), jnp.int32), acc[:, :T - k]], axis=1)
        acc = acc + shifted
    posm_s[...] = jnp.where(mask, acc - 1, -1)


def _fused_body(x_ref, wr_ref, noise_ref, w1_ref, w2_ref, out_ref, aux_ref,
                posm_s, pt_s, xbf_s):
    g = pl.program_id(0)
    T, D = x_ref.shape
    C = (T * TOP_K) // E

    @pl.when(g == 0)
    def _prologue():
        out_ref[...] = jnp.zeros_like(out_ref)
        _router(x_ref[...], wr_ref[...], noise_ref[...],
                posm_s, pt_s, xbf_s, aux_ref)

    iota_c = lax.broadcasted_iota(jnp.int32, (C, T), 0)
    xbf = xbf_s[...]
    cmp_parts = []
    gated_parts = []
    for i in range(G):
        posm_row = posm_s[pl.ds(g * G + i, 1), :]
        pt_row = pt_s[pl.ds(g * G + i, 1), :]
        cmp = iota_c == posm_row
        cmp_parts.append(cmp.astype(jnp.bfloat16))
        gated_parts.append(jnp.where(cmp, pt_row, 0.0).astype(jnp.bfloat16))
    cmp_bf = jnp.concatenate(cmp_parts, axis=0)
    gated_bf = jnp.concatenate(gated_parts, axis=0)

    gathered = jnp.dot(
        cmp_bf, xbf, preferred_element_type=jnp.float32).astype(jnp.bfloat16)

    outs = []
    for i in range(G):
        gi = gathered[i * C:(i + 1) * C, :]
        w1 = w1_ref[i].astype(jnp.bfloat16)
        w2 = w2_ref[i].astype(jnp.bfloat16)
        h = jnp.dot(gi, w1, preferred_element_type=jnp.float32)
        h = jnp.maximum(h, 0.0).astype(jnp.bfloat16)
        outs.append(jnp.dot(h, w2, preferred_element_type=jnp.float32))
    wall = jnp.concatenate(outs, axis=0).astype(jnp.bfloat16)

    out_ref[...] += lax.dot_general(
        gated_bf, wall, (((0,), (0,)), ((), ())),
        preferred_element_type=jnp.float32)


@jax.jit
def kernel(hidden_states, Wr, W1, W2, noise):
    Bs, Ss, D = hidden_states.shape
    T = Bs * Ss
    x = hidden_states.reshape(T, D)

    out, aux = pl.pallas_call(
        _fused_body,
        grid=(E // G,),
        out_shape=(
            jax.ShapeDtypeStruct((T, D), jnp.float32),
            jax.ShapeDtypeStruct((1, 1), jnp.float32),
        ),
        out_specs=(
            pl.BlockSpec((T, D), lambda g: (0, 0)),
            pl.BlockSpec((1, 1), lambda g: (0, 0)),
        ),
        in_specs=[
            pl.BlockSpec((T, D), lambda g: (0, 0)),
            pl.BlockSpec((D, E), lambda g: (0, 0)),
            pl.BlockSpec((T, E), lambda g: (0, 0)),
            pl.BlockSpec((G, D, D), lambda g: (g, 0, 0)),
            pl.BlockSpec((G, D, D), lambda g: (g, 0, 0)),
        ],
        scratch_shapes=[
            pltpu.VMEM((E, T), jnp.int32),
            pltpu.VMEM((E, T), jnp.float32),
            pltpu.VMEM((T, D), jnp.bfloat16),
        ],
        compiler_params=pltpu.CompilerParams(
            dimension_semantics=("arbitrary",)),
    )(x, Wr, noise, W1, W2)

    return out.reshape(Bs, Ss, D), aux.reshape(())


# speedup vs baseline: 1.0558x; 1.0558x over previous
import functools

import jax
import jax.numpy as jnp
from jax import lax
from jax.experimental import pallas as pl
from jax.experimental.pallas import tpu as pltpu

E = 64
TOP_K = 2
G = 2
NBUF = 4


def _router(x, wr, noise, posm_s, pt_s, xbf_s, aux_ref):
    T = x.shape[0]
    C = (T * TOP_K) // E
    logits = jnp.dot(x, wr, preferred_element_type=jnp.float32)
    m = jnp.max(logits, axis=1, keepdims=True)
    p = jnp.exp(logits - m)
    p = p / jnp.sum(p, axis=1, keepdims=True)
    psum = jnp.sum(p, axis=0, keepdims=True)
    aux_ref[...] = (E * C / T) * (jnp.sum(psum, axis=1, keepdims=True) / T)

    nl = jnp.transpose(logits + noise)
    pt_s[...] = jnp.transpose(p)
    xbf_s[...] = x.astype(jnp.bfloat16)

    kbits = lax.bitcast_convert_type(nl, jnp.int32)
    skey = kbits ^ ((kbits >> 31) & jnp.int32(0x7FFFFFFF))

    def count_ge(thr):
        return jnp.sum(jnp.where(skey >= thr, 1, 0), axis=1, keepdims=True)

    def hi_body(_, carry):
        lo, hi = carry
        mid = (lo + hi) >> 1
        ge = count_ge(mid << 16) >= C
        return jnp.where(ge, mid, lo), jnp.where(ge, hi, mid)

    lo, _ = lax.fori_loop(0, 16, hi_body,
                          (jnp.full((E, 1), -32768, jnp.int32),
                           jnp.full((E, 1), 32768, jnp.int32)))
    base = lo << 16

    def lo_body(_, carry):
        lo, hi = carry
        mid = (lo + hi) >> 1
        ge = count_ge(base + mid) >= C
        return jnp.where(ge, mid, lo), jnp.where(ge, hi, mid)

    lo2, _ = lax.fori_loop(0, 16, lo_body,
                           (jnp.zeros((E, 1), jnp.int32),
                            jnp.full((E, 1), 65536, jnp.int32)))
    thr = base + lo2

    mask = skey >= thr
    acc = jnp.where(mask, 1, 0)
    for k in (1, 2, 4, 8, 16, 32, 64, 128, 256, 512, 1024):
        shifted = jnp.concatenate(
            [jnp.zeros((E, k), jnp.int32), acc[:, :T - k]], axis=1)
        acc = acc + shifted
    posm_s[...] = jnp.where(mask, acc - 1, -1)


def _fused_body(x_ref, wr_ref, noise_ref, w1_hbm, w2_hbm, out_ref, aux_ref,
                posm_s, pt_s, xbf_s, w1buf, w2buf, sem):
    g = pl.program_id(0)
    T, D = x_ref.shape
    C = (T * TOP_K) // E
    NG = E // G

    def w_copies(grp, slot):
        H = D // 2
        return tuple(
            pltpu.make_async_copy(
                whbm.at[pl.ds(grp * G, G), pl.ds(h * H, H)],
                wbuf.at[slot, slice(None), pl.ds(h * H, H)],
                sem.at[slot, 2 * j + h])
            for j, (whbm, wbuf) in enumerate(((w1_hbm, w1buf), (w2_hbm, w2buf)))
            for h in range(2))

    @pl.when(g == 0)
    def _prologue():
        for s in range(NBUF):
            for c in w_copies(s, s):
                c.start()
        out_ref[...] = jnp.zeros_like(out_ref)
        _router(x_ref[...], wr_ref[...], noise_ref[...],
                posm_s, pt_s, xbf_s, aux_ref)

    slot = lax.rem(g, NBUF)
    for c in w_copies(g, slot):
        c.wait()

    iota_c = lax.broadcasted_iota(jnp.int32, (C, T), 0)
    xbf = xbf_s[...]
    cmp_parts = []
    gated_parts = []
    for i in range(G):
        posm_row = posm_s[pl.ds(g * G + i, 1), :]
        pt_row = pt_s[pl.ds(g * G + i, 1), :]
        cmp = iota_c == posm_row
        cmp_parts.append(cmp.astype(jnp.bfloat16))
        gated_parts.append(jnp.where(cmp, pt_row, 0.0).astype(jnp.bfloat16))
    cmp_bf = jnp.concatenate(cmp_parts, axis=0)
    gated_bf = jnp.concatenate(gated_parts, axis=0)

    gathered = jnp.dot(
        cmp_bf, xbf, preferred_element_type=jnp.float32).astype(jnp.bfloat16)

    outs = []
    for i in range(G):
        gi = gathered[i * C:(i + 1) * C, :]
        w1 = w1buf[slot, i].astype(jnp.bfloat16)
        w2 = w2buf[slot, i].astype(jnp.bfloat16)
        h = jnp.dot(gi, w1, preferred_element_type=jnp.float32)
        h = jnp.maximum(h, 0.0).astype(jnp.bfloat16)
        outs.append(jnp.dot(h, w2, preferred_element_type=jnp.float32))
    wall = jnp.concatenate(outs, axis=0).astype(jnp.bfloat16)

    out_ref[...] += lax.dot_general(
        gated_bf, wall, (((0,), (0,)), ((), ())),
        preferred_element_type=jnp.float32)

    @pl.when(g + NBUF < NG)
    def _issue_next():
        for c in w_copies(g + NBUF, slot):
            c.start()


@jax.jit
def kernel(hidden_states, Wr, W1, W2, noise):
    Bs, Ss, D = hidden_states.shape
    T = Bs * Ss
    x = hidden_states.reshape(T, D)

    out, aux = pl.pallas_call(
        _fused_body,
        grid=(E // G,),
        out_shape=(
            jax.ShapeDtypeStruct((T, D), jnp.float32),
            jax.ShapeDtypeStruct((1, 1), jnp.float32),
        ),
        out_specs=(
            pl.BlockSpec((T, D), lambda g: (0, 0)),
            pl.BlockSpec((1, 1), lambda g: (0, 0)),
        ),
        in_specs=[
            pl.BlockSpec((T, D), lambda g: (0, 0)),
            pl.BlockSpec((D, E), lambda g: (0, 0)),
            pl.BlockSpec((T, E), lambda g: (0, 0)),
            pl.BlockSpec(memory_space=pl.ANY),
            pl.BlockSpec(memory_space=pl.ANY),
        ],
        scratch_shapes=[
            pltpu.VMEM((E, T), jnp.int32),
            pltpu.VMEM((E, T), jnp.float32),
            pltpu.VMEM((T, D), jnp.bfloat16),
            pltpu.VMEM((NBUF, G, D, D), jnp.float32),
            pltpu.VMEM((NBUF, G, D, D), jnp.float32),
            pltpu.SemaphoreType.DMA((NBUF, 4)),
        ],
        compiler_params=pltpu.CompilerParams(
            dimension_semantics=("arbitrary",)),
    )(x, Wr, noise, W1, W2)

    return out.reshape(Bs, Ss, D), aux.reshape(())


# speedup vs baseline: 1.0561x; 1.0003x over previous
import functools

import jax
import jax.numpy as jnp
from jax import lax
from jax.experimental import pallas as pl
from jax.experimental.pallas import tpu as pltpu

E = 64
TOP_K = 2
G = 2
NBUF = 4


def _router(x, wr, noise, posm_s, pt_s, xbf_s, aux_ref):
    T = x.shape[0]
    C = (T * TOP_K) // E
    logits = jnp.dot(x, wr, preferred_element_type=jnp.float32)
    m = jnp.max(logits, axis=1, keepdims=True)
    p = jnp.exp(logits - m)
    p = p / jnp.sum(p, axis=1, keepdims=True)
    psum = jnp.sum(p, axis=0, keepdims=True)
    aux_ref[...] = (E * C / T) * (jnp.sum(psum, axis=1, keepdims=True) / T)

    nl = jnp.transpose(logits + noise)
    pt_s[...] = jnp.transpose(p)
    xbf_s[...] = x.astype(jnp.bfloat16)

    kbits = lax.bitcast_convert_type(nl, jnp.int32)
    skey = kbits ^ ((kbits >> 31) & jnp.int32(0x7FFFFFFF))

    def count_ge(thr):
        return jnp.sum(jnp.where(skey >= thr, 1, 0), axis=1, keepdims=True)

    def hi_body(_, carry):
        lo, hi = carry
        mid = (lo + hi) >> 1
        ge = count_ge(mid << 16) >= C
        return jnp.where(ge, mid, lo), jnp.where(ge, hi, mid)

    lo, _ = lax.fori_loop(0, 16, hi_body,
                          (jnp.full((E, 1), -32768, jnp.int32),
                           jnp.full((E, 1), 32768, jnp.int32)))
    base = lo << 16

    def lo_body(_, carry):
        lo, hi = carry
        mid = (lo + hi) >> 1
        ge = count_ge(base + mid) >= C
        return jnp.where(ge, mid, lo), jnp.where(ge, hi, mid)

    lo2, _ = lax.fori_loop(0, 16, lo_body,
                           (jnp.zeros((E, 1), jnp.int32),
                            jnp.full((E, 1), 65536, jnp.int32)))
    thr = base + lo2

    mask = skey >= thr
    acc = jnp.where(mask, 1, 0)
    for k in (1, 2, 4, 8, 16, 32, 64, 128, 256, 512, 1024):
        shifted = jnp.concatenate(
            [jnp.zeros((E, k), jnp.int32), acc[:, :T - k]], axis=1)
        acc = acc + shifted
    posm_s[...] = jnp.where(mask, acc - 1, -1)


def _fused_body(x_ref, wr_ref, noise_ref, w1_hbm, w2_hbm, out_ref, aux_ref,
                posm_s, pt_s, xbf_s, w1buf, w2buf, sem):
    g = pl.program_id(0)
    T, D = x_ref.shape
    C = (T * TOP_K) // E
    NG = E // G

    def w_copies(grp, slot):
        H = D // 2
        return tuple(
            pltpu.make_async_copy(
                whbm.at[pl.ds(grp * G + i, 1), pl.ds(h * H, H)],
                wbuf.at[slot, pl.ds(i, 1), pl.ds(h * H, H)],
                sem.at[slot, 4 * j + 2 * i + h])
            for j, (whbm, wbuf) in enumerate(((w1_hbm, w1buf), (w2_hbm, w2buf)))
            for i in range(G)
            for h in range(2))

    @pl.when(g == 0)
    def _prologue():
        for s in range(NBUF):
            for c in w_copies(s, s):
                c.start()
        out_ref[...] = jnp.zeros_like(out_ref)
        _router(x_ref[...], wr_ref[...], noise_ref[...],
                posm_s, pt_s, xbf_s, aux_ref)

    slot = lax.rem(g, NBUF)
    for c in w_copies(g, slot):
        c.wait()

    iota_c = lax.broadcasted_iota(jnp.int32, (C, T), 0)
    xbf = xbf_s[...]
    cmp_parts = []
    gated_parts = []
    for i in range(G):
        posm_row = posm_s[pl.ds(g * G + i, 1), :]
        pt_row = pt_s[pl.ds(g * G + i, 1), :]
        cmp = iota_c == posm_row
        cmp_parts.append(cmp.astype(jnp.bfloat16))
        gated_parts.append(jnp.where(cmp, pt_row, 0.0).astype(jnp.bfloat16))
    cmp_bf = jnp.concatenate(cmp_parts, axis=0)
    gated_bf = jnp.concatenate(gated_parts, axis=0)

    gathered = jnp.dot(
        cmp_bf, xbf, preferred_element_type=jnp.float32).astype(jnp.bfloat16)

    outs = []
    for i in range(G):
        gi = gathered[i * C:(i + 1) * C, :]
        w1 = w1buf[slot, i].astype(jnp.bfloat16)
        w2 = w2buf[slot, i].astype(jnp.bfloat16)
        h = jnp.dot(gi, w1, preferred_element_type=jnp.float32)
        h = jnp.maximum(h, 0.0).astype(jnp.bfloat16)
        outs.append(jnp.dot(h, w2, preferred_element_type=jnp.float32))
    wall = jnp.concatenate(outs, axis=0).astype(jnp.bfloat16)

    out_ref[...] += lax.dot_general(
        gated_bf, wall, (((0,), (0,)), ((), ())),
        preferred_element_type=jnp.float32)

    @pl.when(g + NBUF < NG)
    def _issue_next():
        for c in w_copies(g + NBUF, slot):
            c.start()


@jax.jit
def kernel(hidden_states, Wr, W1, W2, noise):
    Bs, Ss, D = hidden_states.shape
    T = Bs * Ss
    x = hidden_states.reshape(T, D)

    out, aux = pl.pallas_call(
        _fused_body,
        grid=(E // G,),
        out_shape=(
            jax.ShapeDtypeStruct((T, D), jnp.float32),
            jax.ShapeDtypeStruct((1, 1), jnp.float32),
        ),
        out_specs=(
            pl.BlockSpec((T, D), lambda g: (0, 0)),
            pl.BlockSpec((1, 1), lambda g: (0, 0)),
        ),
        in_specs=[
            pl.BlockSpec((T, D), lambda g: (0, 0)),
            pl.BlockSpec((D, E), lambda g: (0, 0)),
            pl.BlockSpec((T, E), lambda g: (0, 0)),
            pl.BlockSpec(memory_space=pl.ANY),
            pl.BlockSpec(memory_space=pl.ANY),
        ],
        scratch_shapes=[
            pltpu.VMEM((E, T), jnp.int32),
            pltpu.VMEM((E, T), jnp.float32),
            pltpu.VMEM((T, D), jnp.bfloat16),
            pltpu.VMEM((NBUF, G, D, D), jnp.float32),
            pltpu.VMEM((NBUF, G, D, D), jnp.float32),
            pltpu.SemaphoreType.DMA((NBUF, 8)),
        ],
        compiler_params=pltpu.CompilerParams(
            dimension_semantics=("arbitrary",)),
    )(x, Wr, noise, W1, W2)

    return out.reshape(Bs, Ss, D), aux.reshape(())
